# Initial kernel scaffold; baseline (speedup 1.0000x reference)
import jax
import jax.numpy as jnp
from jax import lax
from jax.experimental import pallas as pl
from jax.experimental.pallas import tpu as pltpu
from jax.experimental.pallas import tpu_sc as plsc

_N = 10000
_E = 320000
_D = 128
_H = 256
_G = 64

_NC = 2
_NT = 16
_NPAD = 10240
_EPT = _E // _NT
_CH = 80
_NROW = _N // _NT

_mesh = plsc.VectorSubcoreMesh(core_axis_name="c", subcore_axis_name="s")


def _deg_body(ei_c, ei_s, out, degbuf, idxbuf):
    core = lax.axis_index("c")
    sub = lax.axis_index("s")

    @pl.loop(0, _NPAD // 16)
    def _zero(i):
        degbuf[pl.ds(i * 16, 16)] = jnp.zeros((16,), jnp.float32)

    ones = jnp.ones((16,), jnp.float32)
    blk = 2000

    def branch_pass(ei):
        @pl.loop(0, _EPT // blk)
        def _blkloop(b):
            pltpu.sync_copy(ei.at[1, pl.ds(sub * _EPT + b * blk, blk)], idxbuf)

            @pl.loop(0, blk // 16)
            def _inner(j):
                idx = idxbuf[pl.ds(j * 16, 16)]
                plsc.addupdate_scatter(degbuf, [idx], ones)

    @pl.when(core == 0)
    def _c0():
        branch_pass(ei_c)

    @pl.when(core == 1)
    def _c1():
        branch_pass(ei_s)

    pltpu.sync_copy(degbuf, out.at[core, sub])


def _deg_call(ei_c, ei_s):
    return pl.kernel(
        _deg_body,
        out_type=jax.ShapeDtypeStruct((_NC, _NT, _NPAD), jnp.float32),
        mesh=_mesh,
        scratch_types=[
            pltpu.VMEM((_NPAD,), jnp.float32),
            pltpu.VMEM((2000,), jnp.int32),
        ],
    )(ei_c, ei_s)


def _zero_zbuf(zeros):
    @pl.loop(0, _NROW)
    def _r(r):
        @pl.loop(0, _D // 16)
        def _c(j):
            zeros[r, pl.ds(j * 16, 16)] = jnp.zeros((16,), jnp.float32)


def _zero_acc(sub, zeros, acc):
    pltpu.sync_copy(zeros, acc.at[pl.ds(sub * _NROW, _NROW)])


def _edge_pass(sub, tbl, ei, acc, idx_s, idx_d, rows):
    @pl.loop(0, _EPT // _CH)
    def _chunk(i):
        off = sub * _EPT + i * _CH
        pltpu.sync_copy(ei.at[0, pl.ds(off, _CH)], idx_s)
        pltpu.sync_copy(ei.at[1, pl.ds(off, _CH)], idx_d)
        pltpu.sync_copy(tbl.at[idx_s], rows)
        pltpu.sync_copy(rows, acc.at[idx_d], add=True)


def _flush(sub, acc, out):
    sl = pl.ds(sub * _NROW, _NROW)
    pltpu.sync_copy(acc.at[sl], out.at[sl])


def _edge_scratch():
    return [
        pltpu.VMEM_SHARED((_N, _D), jnp.float32),
        pltpu.VMEM((_NROW, _D), jnp.float32),
        pltpu.VMEM((_CH,), jnp.int32),
        pltpu.VMEM((_CH,), jnp.int32),
        pltpu.VMEM((_CH, _D), jnp.float32),
    ]


def _conv1_body(ei_c, ei_s, tbl_c, tbl_s, out_c, out_s,
                acc, zeros, idx_s, idx_d, rows):
    core = lax.axis_index("c")
    sub = lax.axis_index("s")
    _zero_zbuf(zeros)
    _zero_acc(sub, zeros, acc)
    plsc.subcore_barrier()

    @pl.when(core == 0)
    def _c0():
        _edge_pass(sub, tbl_c, ei_c, acc, idx_s, idx_d, rows)

    @pl.when(core == 1)
    def _c1():
        _edge_pass(sub, tbl_s, ei_s, acc, idx_s, idx_d, rows)

    plsc.subcore_barrier()

    @pl.when(core == 0)
    def _f0():
        _flush(sub, acc, out_c)

    @pl.when(core == 1)
    def _f1():
        _flush(sub, acc, out_s)


def _conv1_call(ei_c, ei_s, tbl_c, tbl_s):
    shp = jax.ShapeDtypeStruct((_N, _D), jnp.float32)
    return pl.kernel(
        _conv1_body,
        out_type=(shp, shp),
        mesh=_mesh,
        scratch_types=_edge_scratch(),
    )(ei_c, ei_s, tbl_c, tbl_s)


def _conv2_body(ei_c, ei_s, tc0, tc1, ts0, ts1, oc0, oc1, os0, os1,
                acc, zeros, idx_s, idx_d, rows):
    core = lax.axis_index("c")
    sub = lax.axis_index("s")
    _zero_zbuf(zeros)
    _zero_acc(sub, zeros, acc)
    plsc.subcore_barrier()

    @pl.when(core == 0)
    def _c0():
        _edge_pass(sub, tc0, ei_c, acc, idx_s, idx_d, rows)

    @pl.when(core == 1)
    def _c1():
        _edge_pass(sub, tc1, ei_c, acc, idx_s, idx_d, rows)

    plsc.subcore_barrier()

    @pl.when(core == 0)
    def _f0():
        _flush(sub, acc, oc0)

    @pl.when(core == 1)
    def _f1():
        _flush(sub, acc, oc1)

    _zero_acc(sub, zeros, acc)
    plsc.subcore_barrier()

    @pl.when(core == 0)
    def _s0():
        _edge_pass(sub, ts0, ei_s, acc, idx_s, idx_d, rows)

    @pl.when(core == 1)
    def _s1():
        _edge_pass(sub, ts1, ei_s, acc, idx_s, idx_d, rows)

    plsc.subcore_barrier()

    @pl.when(core == 0)
    def _g0():
        _flush(sub, acc, os0)

    @pl.when(core == 1)
    def _g1():
        _flush(sub, acc, os1)


def _conv2_call(ei_c, ei_s, tc0, tc1, ts0, ts1):
    shp = jax.ShapeDtypeStruct((_N, _D), jnp.float32)
    return pl.kernel(
        _conv2_body,
        out_type=(shp, shp, shp, shp),
        mesh=_mesh,
        scratch_types=_edge_scratch(),
    )(ei_c, ei_s, tc0, tc1, ts0, ts1)


def _prep_body(degp_ref, xc_ref, xs_ref, dinv_ref, xsc_ref, xss_ref):
    deg = jnp.sum(degp_ref[...], axis=1) + 1.0
    dinv = lax.rsqrt(deg)
    dinv_ref[...] = dinv
    xsc_ref[...] = xc_ref[...] * dinv[0, :_N, None]
    xss_ref[...] = xs_ref[...] * dinv[1, :_N, None]


def _prep_call(deg_parts, x_c, x_s):
    return pl.pallas_call(
        _prep_body,
        out_shape=(
            jax.ShapeDtypeStruct((_NC, _NPAD), jnp.float32),
            jax.ShapeDtypeStruct((_N, _D), jnp.float32),
            jax.ShapeDtypeStruct((_N, _D), jnp.float32),
        ),
    )(deg_parts, x_c, x_s)


def _mid_body(agg_ref, xs_ref, dinv_ref, w_ref, b_ref, g_ref, be_ref,
              h0_ref, h1_ref):
    dinv = dinv_ref[...]
    t = (agg_ref[...] + xs_ref[...]) * dinv
    z = jnp.dot(t, w_ref[...], preferred_element_type=jnp.float32) + b_ref[...]
    m = jnp.mean(z, axis=0, keepdims=True)
    v = jnp.mean((z - m) * (z - m), axis=0, keepdims=True)
    h = jnp.maximum((z - m) * lax.rsqrt(v + 1e-5) * g_ref[...] + be_ref[...],
                    0.0)
    hs = h * dinv
    h0_ref[...] = hs[:, :_D]
    h1_ref[...] = hs[:, _D:]


def _mid_call(agg, xs, dinv, w, b, g, be):
    shp = jax.ShapeDtypeStruct((_N, _D), jnp.float32)
    return pl.pallas_call(
        _mid_body,
        out_shape=(shp, shp),
    )(agg, xs, dinv, w, b, g, be)


def _post_body(a0_ref, a1_ref, h0_ref, h1_ref, dinv_ref, w0_ref, w1_ref,
               b_ref, g_ref, be_ref, batch_ref, pool_ref):
    dinv = dinv_ref[...]
    t0 = (a0_ref[...] + h0_ref[...]) * dinv
    t1 = (a1_ref[...] + h1_ref[...]) * dinv
    z = (jnp.dot(t0, w0_ref[...], preferred_element_type=jnp.float32)
         + jnp.dot(t1, w1_ref[...], preferred_element_type=jnp.float32)
         + b_ref[...])
    m = jnp.mean(z, axis=0, keepdims=True)
    v = jnp.mean((z - m) * (z - m), axis=0, keepdims=True)
    h = jnp.maximum((z - m) * lax.rsqrt(v + 1e-5) * g_ref[...] + be_ref[...],
                    0.0)
    seg = batch_ref[...]
    onehot = (lax.broadcasted_iota(jnp.int32, (_G, _N), 0)
              == seg).astype(jnp.float32)
    s = jnp.dot(onehot, h, preferred_element_type=jnp.float32)
    cnt = jnp.sum(onehot, axis=1, keepdims=True)
    pool_ref[...] = s / jnp.maximum(cnt, 1.0)


def _post_call(a0, a1, h0, h1, dinv, w0, w1, b, g, be, batch):
    return pl.pallas_call(
        _post_body,
        out_shape=jax.ShapeDtypeStruct((_G, _H), jnp.float32),
    )(a0, a1, h0, h1, dinv, w0, w1, b, g, be, batch)


def _head_body(hc_ref, hs_ref, w1_ref, b1_ref, w2_ref, b2_ref, out_ref):
    x = jnp.concatenate([hc_ref[...], hs_ref[...]], axis=1)
    x = jnp.maximum(
        jnp.dot(x, w1_ref[...], preferred_element_type=jnp.float32)
        + b1_ref[...], 0.0)
    out_ref[...] = (jnp.dot(x, w2_ref[...], preferred_element_type=jnp.float32)
                    + b2_ref[...])


def _head_call(hc, hs, w1, b1, w2, b2):
    return pl.pallas_call(
        _head_body,
        out_shape=jax.ShapeDtypeStruct((_G, 2), jnp.float32),
    )(hc, hs, w1, b1, w2, b2)


def kernel(x_c, edge_index_c, batch_c, x_s, edge_index_s, batch_s,
           Wc1, bc1, gc1, betac1, Wc2, bc2, gc2, betac2,
           Ws1, bs1, gs1, betas1, Ws2, bs2, gs2, betas2,
           W_fc1, b_fc1, W_fc2, b_fc2):
    ei_c = edge_index_c.astype(jnp.int32)
    ei_s = edge_index_s.astype(jnp.int32)

    deg_parts = _deg_call(ei_c, ei_s)
    dinv2, xsc, xss = _prep_call(deg_parts, x_c, x_s)
    dinv_c = dinv2[0, :_N, None]
    dinv_s = dinv2[1, :_N, None]

    agg1_c, agg1_s = _conv1_call(ei_c, ei_s, xsc, xss)

    hc0, hc1 = _mid_call(agg1_c, xsc, dinv_c, Wc1, bc1.reshape(1, _H),
                         gc1.reshape(1, _H), betac1.reshape(1, _H))
    hs0, hs1 = _mid_call(agg1_s, xss, dinv_s, Ws1, bs1.reshape(1, _H),
                         gs1.reshape(1, _H), betas1.reshape(1, _H))

    ac0, ac1, as0, as1 = _conv2_call(ei_c, ei_s, hc0, hc1, hs0, hs1)

    pool_c = _post_call(ac0, ac1, hc0, hc1, dinv_c,
                        Wc2[:_D], Wc2[_D:], bc2.reshape(1, _H),
                        gc2.reshape(1, _H), betac2.reshape(1, _H),
                        batch_c.astype(jnp.int32).reshape(1, _N))
    pool_s = _post_call(as0, as1, hs0, hs1, dinv_s,
                        Ws2[:_D], Ws2[_D:], bs2.reshape(1, _H),
                        gs2.reshape(1, _H), betas2.reshape(1, _H),
                        batch_s.astype(jnp.int32).reshape(1, _N))

    return _head_call(pool_c, pool_s, W_fc1, b_fc1.reshape(1, _H),
                      W_fc2, b_fc2.reshape(1, 2))


# speedup vs baseline: 10.3975x; 10.3975x over previous
import jax
import jax.numpy as jnp
from jax import lax
from jax.experimental import pallas as pl
from jax.experimental.pallas import tpu as pltpu
from jax.experimental.pallas import tpu_sc as plsc

_N = 10000
_E = 320000
_D = 128
_H = 256
_G = 64

_NC = 2
_NT = 16
_NPAD = 10240
_EPT = _E // _NT
_CH = 80
_NROW = _NPAD // _NT

_mesh = plsc.VectorSubcoreMesh(core_axis_name="c", subcore_axis_name="s")
_sc_params = pltpu.CompilerParams(needs_layout_passes=False, use_tc_tiling_on_sc=False)


def _deg_body(dst_c, dst_s, out, degbuf, idxbuf):
    core = lax.axis_index("c")
    sub = lax.axis_index("s")

    @pl.loop(0, _NPAD // 16)
    def _zero(i):
        degbuf[pl.ds(i * 16, 16)] = jnp.zeros((16,), jnp.float32)

    ones = jnp.ones((16,), jnp.float32)
    blk = 2000

    def branch_pass(dst):
        @pl.loop(0, _EPT // blk)
        def _blkloop(b):
            pltpu.sync_copy(dst.at[pl.ds(sub * _EPT + b * blk, blk)], idxbuf)

            @pl.loop(0, blk // 16)
            def _inner(j):
                idx = idxbuf[pl.ds(j * 16, 16)]
                plsc.addupdate_scatter(degbuf, [idx], ones)

    @pl.when(core == 0)
    def _c0():
        branch_pass(dst_c)

    @pl.when(core == 1)
    def _c1():
        branch_pass(dst_s)

    pltpu.sync_copy(degbuf, out.at[core, sub])


def _deg_call(dst_c, dst_s):
    return pl.kernel(
        _deg_body,
        out_type=jax.ShapeDtypeStruct((_NC, _NT, _NPAD), jnp.float32),
        mesh=_mesh,
        compiler_params=_sc_params,
        scratch_types=[
            pltpu.VMEM((_NPAD,), jnp.float32),
            pltpu.VMEM((2000,), jnp.int32),
        ],
    )(dst_c, dst_s)


def _zero_rows(rows):
    @pl.loop(0, _CH)
    def _r(r):
        @pl.loop(0, _D // 16)
        def _c(j):
            rows[r, pl.ds(j * 16, 16)] = jnp.zeros((16,), jnp.float32)


def _zero_acc(sub, rows, acc):
    @pl.loop(0, _NROW // _CH)
    def _k(k):
        pltpu.sync_copy(rows, acc.at[pl.ds(sub * _NROW + k * _CH, _CH)])


def _edge_pass(sub, tbl, src, dst, acc, idx_s, idx_d, rows):
    @pl.loop(0, _EPT // _CH)
    def _chunk(i):
        off = sub * _EPT + i * _CH
        pltpu.sync_copy(src.at[pl.ds(off, _CH)], idx_s)
        pltpu.sync_copy(dst.at[pl.ds(off, _CH)], idx_d)
        pltpu.sync_copy(tbl.at[idx_s], rows)
        pltpu.sync_copy(rows, acc.at[idx_d], add=True)


def _flush(sub, acc, out):
    sl = pl.ds(sub * _NROW, _NROW)
    pltpu.sync_copy(acc.at[sl], out.at[sl])


def _edge_scratch():
    return [
        pltpu.VMEM_SHARED((_NPAD, _D), jnp.float32),
        pltpu.VMEM((_CH,), jnp.int32),
        pltpu.VMEM((_CH,), jnp.int32),
        pltpu.VMEM((_CH, _D), jnp.float32),
    ]


def _conv1_body(src_c, dst_c, src_s, dst_s, tbl_c, tbl_s, out_c, out_s,
                acc, idx_s, idx_d, rows):
    core = lax.axis_index("c")
    sub = lax.axis_index("s")
    _zero_rows(rows)
    _zero_acc(sub, rows, acc)
    plsc.subcore_barrier()

    @pl.when(core == 0)
    def _c0():
        _edge_pass(sub, tbl_c, src_c, dst_c, acc, idx_s, idx_d, rows)

    @pl.when(core == 1)
    def _c1():
        _edge_pass(sub, tbl_s, src_s, dst_s, acc, idx_s, idx_d, rows)

    plsc.subcore_barrier()

    @pl.when(core == 0)
    def _f0():
        _flush(sub, acc, out_c)

    @pl.when(core == 1)
    def _f1():
        _flush(sub, acc, out_s)


def _conv1_call(src_c, dst_c, src_s, dst_s, tbl_c, tbl_s):
    shp = jax.ShapeDtypeStruct((_NPAD, _D), jnp.float32)
    return pl.kernel(
        _conv1_body,
        out_type=(shp, shp),
        mesh=_mesh,
        compiler_params=_sc_params,
        scratch_types=_edge_scratch(),
    )(src_c, dst_c, src_s, dst_s, tbl_c, tbl_s)


def _conv2_body(src_c, dst_c, src_s, dst_s, tc0, tc1, ts0, ts1,
                oc0, oc1, os0, os1, acc, idx_s, idx_d, rows):
    core = lax.axis_index("c")
    sub = lax.axis_index("s")
    _zero_rows(rows)
    _zero_acc(sub, rows, acc)
    plsc.subcore_barrier()

    @pl.when(core == 0)
    def _c0():
        _edge_pass(sub, tc0, src_c, dst_c, acc, idx_s, idx_d, rows)

    @pl.when(core == 1)
    def _c1():
        _edge_pass(sub, tc1, src_c, dst_c, acc, idx_s, idx_d, rows)

    plsc.subcore_barrier()

    @pl.when(core == 0)
    def _f0():
        _flush(sub, acc, oc0)

    @pl.when(core == 1)
    def _f1():
        _flush(sub, acc, oc1)

    _zero_rows(rows)
    _zero_acc(sub, rows, acc)
    plsc.subcore_barrier()

    @pl.when(core == 0)
    def _s0():
        _edge_pass(sub, ts0, src_s, dst_s, acc, idx_s, idx_d, rows)

    @pl.when(core == 1)
    def _s1():
        _edge_pass(sub, ts1, src_s, dst_s, acc, idx_s, idx_d, rows)

    plsc.subcore_barrier()

    @pl.when(core == 0)
    def _g0():
        _flush(sub, acc, os0)

    @pl.when(core == 1)
    def _g1():
        _flush(sub, acc, os1)


def _conv2_call(src_c, dst_c, src_s, dst_s, tc0, tc1, ts0, ts1):
    shp = jax.ShapeDtypeStruct((_NPAD, _D), jnp.float32)
    return pl.kernel(
        _conv2_body,
        out_type=(shp, shp, shp, shp),
        mesh=_mesh,
        compiler_params=_sc_params,
        scratch_types=_edge_scratch(),
    )(src_c, dst_c, src_s, dst_s, tc0, tc1, ts0, ts1)


def _prep_body(degp_ref, xc_ref, xs_ref, dinv_ref, xsc_ref, xss_ref):
    deg = jnp.sum(degp_ref[...], axis=1) + 1.0
    dinv = lax.rsqrt(deg)
    dinv_ref[...] = dinv
    pad = jnp.zeros((_NPAD - _N, _D), jnp.float32)
    xsc_ref[:_N] = xc_ref[...] * dinv[0, :_N, None]
    xsc_ref[_N:] = pad
    xss_ref[:_N] = xs_ref[...] * dinv[1, :_N, None]
    xss_ref[_N:] = pad


def _prep_call(deg_parts, x_c, x_s):
    return pl.pallas_call(
        _prep_body,
        out_shape=(
            jax.ShapeDtypeStruct((_NC, _NPAD), jnp.float32),
            jax.ShapeDtypeStruct((_NPAD, _D), jnp.float32),
            jax.ShapeDtypeStruct((_NPAD, _D), jnp.float32),
        ),
    )(deg_parts, x_c, x_s)


def _mid_body(agg_ref, xs_ref, dinv_ref, w_ref, b_ref, g_ref, be_ref,
              h0_ref, h1_ref):
    dinv = dinv_ref[...]
    t = (agg_ref[:_N] + xs_ref[:_N]) * dinv
    z = jnp.dot(t, w_ref[...], preferred_element_type=jnp.float32) + b_ref[...]
    m = jnp.mean(z, axis=0, keepdims=True)
    v = jnp.mean((z - m) * (z - m), axis=0, keepdims=True)
    h = jnp.maximum((z - m) * lax.rsqrt(v + 1e-5) * g_ref[...] + be_ref[...],
                    0.0)
    hs = h * dinv
    pad = jnp.zeros((_NPAD - _N, _D), jnp.float32)
    h0_ref[:_N] = hs[:, :_D]
    h0_ref[_N:] = pad
    h1_ref[:_N] = hs[:, _D:]
    h1_ref[_N:] = pad


def _mid_call(agg, xs, dinv, w, b, g, be):
    shp = jax.ShapeDtypeStruct((_NPAD, _D), jnp.float32)
    return pl.pallas_call(
        _mid_body,
        out_shape=(shp, shp),
    )(agg, xs, dinv, w, b, g, be)


def _post_body(a0_ref, a1_ref, h0_ref, h1_ref, dinv_ref, w0_ref, w1_ref,
               b_ref, g_ref, be_ref, batch_ref, pool_ref):
    dinv = dinv_ref[...]
    t0 = (a0_ref[:_N] + h0_ref[:_N]) * dinv
    t1 = (a1_ref[:_N] + h1_ref[:_N]) * dinv
    z = (jnp.dot(t0, w0_ref[...], preferred_element_type=jnp.float32)
         + jnp.dot(t1, w1_ref[...], preferred_element_type=jnp.float32)
         + b_ref[...])
    m = jnp.mean(z, axis=0, keepdims=True)
    v = jnp.mean((z - m) * (z - m), axis=0, keepdims=True)
    h = jnp.maximum((z - m) * lax.rsqrt(v + 1e-5) * g_ref[...] + be_ref[...],
                    0.0)
    seg = batch_ref[...]
    onehot = (lax.broadcasted_iota(jnp.int32, (_G, _N), 0)
              == seg).astype(jnp.float32)
    s = jnp.dot(onehot, h, preferred_element_type=jnp.float32)
    cnt = jnp.sum(onehot, axis=1, keepdims=True)
    pool_ref[...] = s / jnp.maximum(cnt, 1.0)


def _post_call(a0, a1, h0, h1, dinv, w0, w1, b, g, be, batch):
    return pl.pallas_call(
        _post_body,
        out_shape=jax.ShapeDtypeStruct((_G, _H), jnp.float32),
    )(a0, a1, h0, h1, dinv, w0, w1, b, g, be, batch)


def _head_body(hc_ref, hs_ref, w1_ref, b1_ref, w2_ref, b2_ref, out_ref):
    x = jnp.concatenate([hc_ref[...], hs_ref[...]], axis=1)
    x = jnp.maximum(
        jnp.dot(x, w1_ref[...], preferred_element_type=jnp.float32)
        + b1_ref[...], 0.0)
    out_ref[...] = (jnp.dot(x, w2_ref[...], preferred_element_type=jnp.float32)
                    + b2_ref[...])


def _head_call(hc, hs, w1, b1, w2, b2):
    return pl.pallas_call(
        _head_body,
        out_shape=jax.ShapeDtypeStruct((_G, 2), jnp.float32),
    )(hc, hs, w1, b1, w2, b2)


def kernel(x_c, edge_index_c, batch_c, x_s, edge_index_s, batch_s,
           Wc1, bc1, gc1, betac1, Wc2, bc2, gc2, betac2,
           Ws1, bs1, gs1, betas1, Ws2, bs2, gs2, betas2,
           W_fc1, b_fc1, W_fc2, b_fc2):
    src_c, dst_c = edge_index_c[0].astype(jnp.int32), edge_index_c[1].astype(jnp.int32)
    src_s, dst_s = edge_index_s[0].astype(jnp.int32), edge_index_s[1].astype(jnp.int32)

    deg_parts = _deg_call(dst_c, dst_s)
    dinv2, xsc, xss = _prep_call(deg_parts, x_c, x_s)
    dinv_c = dinv2[0, :_N, None]
    dinv_s = dinv2[1, :_N, None]

    agg1_c, agg1_s = _conv1_call(src_c, dst_c, src_s, dst_s, xsc, xss)

    hc0, hc1 = _mid_call(agg1_c, xsc, dinv_c, Wc1, bc1.reshape(1, _H),
                         gc1.reshape(1, _H), betac1.reshape(1, _H))
    hs0, hs1 = _mid_call(agg1_s, xss, dinv_s, Ws1, bs1.reshape(1, _H),
                         gs1.reshape(1, _H), betas1.reshape(1, _H))

    ac0, ac1, as0, as1 = _conv2_call(src_c, dst_c, src_s, dst_s,
                                     hc0, hc1, hs0, hs1)

    pool_c = _post_call(ac0, ac1, hc0, hc1, dinv_c,
                        Wc2[:_D], Wc2[_D:], bc2.reshape(1, _H),
                        gc2.reshape(1, _H), betac2.reshape(1, _H),
                        batch_c.astype(jnp.int32).reshape(1, _N))
    pool_s = _post_call(as0, as1, hs0, hs1, dinv_s,
                        Ws2[:_D], Ws2[_D:], bs2.reshape(1, _H),
                        gs2.reshape(1, _H), betas2.reshape(1, _H),
                        batch_s.astype(jnp.int32).reshape(1, _N))

    return _head_call(pool_c, pool_s, W_fc1, b_fc1.reshape(1, _H),
                      W_fc2, b_fc2.reshape(1, 2))


# speedup vs baseline: 25.3769x; 2.4407x over previous
import jax
import jax.numpy as jnp
from jax import lax
from jax.experimental import pallas as pl
from jax.experimental.pallas import tpu as pltpu
from jax.experimental.pallas import tpu_sc as plsc

_N = 10000
_E = 320000
_D = 128
_H = 256
_G = 64

_NC = 2
_NT = 16
_NPAD = 10240
_EPT = _E // _NT
_W = 100
_B = 40
_CPT = _EPT // _W
_NBLK = _CPT // _B
_NROW = _NPAD // _NT

_mesh = plsc.VectorSubcoreMesh(core_axis_name="c", subcore_axis_name="s")
_sc_params = pltpu.CompilerParams(needs_layout_passes=False, use_tc_tiling_on_sc=False)


def _deg_body(dst_c, dst_s, out, degbuf, idxbuf):
    core = lax.axis_index("c")
    sub = lax.axis_index("s")

    @pl.loop(0, _NPAD // 16)
    def _zero(i):
        degbuf[pl.ds(i * 16, 16)] = jnp.zeros((16,), jnp.float32)

    ones = jnp.ones((16,), jnp.float32)
    blk = 2000

    def branch_pass(dst):
        @pl.loop(0, _EPT // blk)
        def _blkloop(b):
            pltpu.sync_copy(dst.at[pl.ds(sub * _EPT + b * blk, blk)], idxbuf)

            @pl.loop(0, blk // 16)
            def _inner(j):
                idx = idxbuf[pl.ds(j * 16, 16)]
                plsc.addupdate_scatter(degbuf, [idx], ones)

    @pl.when(core == 0)
    def _c0():
        branch_pass(dst_c)

    @pl.when(core == 1)
    def _c1():
        branch_pass(dst_s)

    pltpu.sync_copy(degbuf, out.at[core, sub])


def _deg_call(dst_c, dst_s):
    return pl.kernel(
        _deg_body,
        out_type=jax.ShapeDtypeStruct((_NC, _NT, _NPAD), jnp.float32),
        mesh=_mesh,
        compiler_params=_sc_params,
        scratch_types=[
            pltpu.VMEM((_NPAD,), jnp.float32),
            pltpu.VMEM((2000,), jnp.int32),
        ],
    )(dst_c, dst_s)


def _zero_rows(rows):
    @pl.loop(0, _W)
    def _r(r):
        @pl.loop(0, _D // 16)
        def _c(j):
            rows[r, pl.ds(j * 16, 16)] = jnp.zeros((16,), jnp.float32)


def _zero_acc(sub, rows, acc):
    @pl.loop(0, _NROW // 80)
    def _k(k):
        pltpu.sync_copy(rows.at[pl.ds(0, 80)],
                        acc.at[pl.ds(sub * _NROW + k * 80, 80)])


def _edge_pass(sub, tbl, src2, dst2, acc, idxs, idxd, rows0, rows1,
               sem0, sem1):
    npairs = _B // 2

    def wait_g(rows, sem):
        pltpu.make_async_copy(tbl.at[idxs.at[0]], rows, sem).wait()

    @pl.loop(0, _NBLK)
    def _blk(b):
        base = sub * _CPT + b * _B
        pltpu.sync_copy(src2.at[pl.ds(base, _B)], idxs)
        pltpu.sync_copy(dst2.at[pl.ds(base, _B)], idxd)
        pltpu.async_copy(tbl.at[idxs.at[0]], rows0, sem0)

        @pl.loop(0, npairs)
        def _pair(j):
            d1 = pltpu.async_copy(tbl.at[idxs.at[2 * j + 1]], rows1, sem1)
            wait_g(rows0, sem0)
            pltpu.sync_copy(rows0, acc.at[idxd.at[2 * j]], add=True)

            @pl.when(j < npairs - 1)
            def _nxt():
                pltpu.async_copy(tbl.at[idxs.at[2 * j + 2]], rows0, sem0)

            d1.wait()
            pltpu.sync_copy(rows1, acc.at[idxd.at[2 * j + 1]], add=True)


def _flush(sub, acc, out):
    sl = pl.ds(sub * _NROW, _NROW)
    pltpu.sync_copy(acc.at[sl], out.at[sl])


def _edge_scratch():
    return [
        pltpu.VMEM_SHARED((_NPAD, _D), jnp.float32),
        pltpu.VMEM((_B, _W), jnp.int32),
        pltpu.VMEM((_B, _W), jnp.int32),
        pltpu.VMEM((_W, _D), jnp.float32),
        pltpu.VMEM((_W, _D), jnp.float32),
        pltpu.SemaphoreType.DMA,
        pltpu.SemaphoreType.DMA,
    ]


def _conv1_body(src_c, dst_c, src_s, dst_s, tbl_c, tbl_s, out_c, out_s,
                acc, idxs, idxd, rows0, rows1, sem0, sem1):
    core = lax.axis_index("c")
    sub = lax.axis_index("s")
    _zero_rows(rows0)
    _zero_acc(sub, rows0, acc)
    plsc.subcore_barrier()

    @pl.when(core == 0)
    def _c0():
        _edge_pass(sub, tbl_c, src_c, dst_c, acc, idxs, idxd, rows0, rows1,
                   sem0, sem1)

    @pl.when(core == 1)
    def _c1():
        _edge_pass(sub, tbl_s, src_s, dst_s, acc, idxs, idxd, rows0, rows1,
                   sem0, sem1)

    plsc.subcore_barrier()

    @pl.when(core == 0)
    def _f0():
        _flush(sub, acc, out_c)

    @pl.when(core == 1)
    def _f1():
        _flush(sub, acc, out_s)


def _conv1_call(src_c, dst_c, src_s, dst_s, tbl_c, tbl_s):
    shp = jax.ShapeDtypeStruct((_NPAD, _D), jnp.float32)
    return pl.kernel(
        _conv1_body,
        out_type=(shp, shp),
        mesh=_mesh,
        compiler_params=_sc_params,
        scratch_types=_edge_scratch(),
    )(src_c, dst_c, src_s, dst_s, tbl_c, tbl_s)


def _conv2_body(src_c, dst_c, src_s, dst_s, tc0, tc1, ts0, ts1,
                oc0, oc1, os0, os1, acc, idxs, idxd, rows0, rows1,
                sem0, sem1):
    core = lax.axis_index("c")
    sub = lax.axis_index("s")
    _zero_rows(rows0)
    _zero_acc(sub, rows0, acc)
    plsc.subcore_barrier()

    @pl.when(core == 0)
    def _c0():
        _edge_pass(sub, tc0, src_c, dst_c, acc, idxs, idxd, rows0, rows1,
                   sem0, sem1)

    @pl.when(core == 1)
    def _c1():
        _edge_pass(sub, tc1, src_c, dst_c, acc, idxs, idxd, rows0, rows1,
                   sem0, sem1)

    plsc.subcore_barrier()

    @pl.when(core == 0)
    def _f0():
        _flush(sub, acc, oc0)

    @pl.when(core == 1)
    def _f1():
        _flush(sub, acc, oc1)

    _zero_rows(rows0)
    _zero_acc(sub, rows0, acc)
    plsc.subcore_barrier()

    @pl.when(core == 0)
    def _s0():
        _edge_pass(sub, ts0, src_s, dst_s, acc, idxs, idxd, rows0, rows1,
                   sem0, sem1)

    @pl.when(core == 1)
    def _s1():
        _edge_pass(sub, ts1, src_s, dst_s, acc, idxs, idxd, rows0, rows1,
                   sem0, sem1)

    plsc.subcore_barrier()

    @pl.when(core == 0)
    def _g0():
        _flush(sub, acc, os0)

    @pl.when(core == 1)
    def _g1():
        _flush(sub, acc, os1)


def _conv2_call(src_c, dst_c, src_s, dst_s, tc0, tc1, ts0, ts1):
    shp = jax.ShapeDtypeStruct((_NPAD, _D), jnp.float32)
    return pl.kernel(
        _conv2_body,
        out_type=(shp, shp, shp, shp),
        mesh=_mesh,
        compiler_params=_sc_params,
        scratch_types=_edge_scratch(),
    )(src_c, dst_c, src_s, dst_s, tc0, tc1, ts0, ts1)


def _prep_body(degp_ref, xc_ref, xs_ref, dinv_ref, xsc_ref, xss_ref):
    deg = jnp.sum(degp_ref[...], axis=1) + 1.0
    dinv = lax.rsqrt(deg)
    dinv_ref[...] = dinv
    pad = jnp.zeros((_NPAD - _N, _D), jnp.float32)
    xsc_ref[:_N] = xc_ref[...] * dinv[0, :_N, None]
    xsc_ref[_N:] = pad
    xss_ref[:_N] = xs_ref[...] * dinv[1, :_N, None]
    xss_ref[_N:] = pad


def _prep_call(deg_parts, x_c, x_s):
    return pl.pallas_call(
        _prep_body,
        out_shape=(
            jax.ShapeDtypeStruct((_NC, _NPAD), jnp.float32),
            jax.ShapeDtypeStruct((_NPAD, _D), jnp.float32),
            jax.ShapeDtypeStruct((_NPAD, _D), jnp.float32),
        ),
    )(deg_parts, x_c, x_s)


def _mid_body(agg_ref, xs_ref, dinv_ref, w_ref, b_ref, g_ref, be_ref,
              h0_ref, h1_ref):
    dinv = dinv_ref[...]
    t = (agg_ref[:_N] + xs_ref[:_N]) * dinv
    z = jnp.dot(t, w_ref[...], preferred_element_type=jnp.float32) + b_ref[...]
    m = jnp.mean(z, axis=0, keepdims=True)
    v = jnp.mean((z - m) * (z - m), axis=0, keepdims=True)
    h = jnp.maximum((z - m) * lax.rsqrt(v + 1e-5) * g_ref[...] + be_ref[...],
                    0.0)
    hs = h * dinv
    pad = jnp.zeros((_NPAD - _N, _D), jnp.float32)
    h0_ref[:_N] = hs[:, :_D]
    h0_ref[_N:] = pad
    h1_ref[:_N] = hs[:, _D:]
    h1_ref[_N:] = pad


def _mid_call(agg, xs, dinv, w, b, g, be):
    shp = jax.ShapeDtypeStruct((_NPAD, _D), jnp.float32)
    return pl.pallas_call(
        _mid_body,
        out_shape=(shp, shp),
    )(agg, xs, dinv, w, b, g, be)


def _post_body(a0_ref, a1_ref, h0_ref, h1_ref, dinv_ref, w0_ref, w1_ref,
               b_ref, g_ref, be_ref, batch_ref, pool_ref):
    dinv = dinv_ref[...]
    t0 = (a0_ref[:_N] + h0_ref[:_N]) * dinv
    t1 = (a1_ref[:_N] + h1_ref[:_N]) * dinv
    z = (jnp.dot(t0, w0_ref[...], preferred_element_type=jnp.float32)
         + jnp.dot(t1, w1_ref[...], preferred_element_type=jnp.float32)
         + b_ref[...])
    m = jnp.mean(z, axis=0, keepdims=True)
    v = jnp.mean((z - m) * (z - m), axis=0, keepdims=True)
    h = jnp.maximum((z - m) * lax.rsqrt(v + 1e-5) * g_ref[...] + be_ref[...],
                    0.0)
    seg = batch_ref[...]
    onehot = (lax.broadcasted_iota(jnp.int32, (_G, _N), 0)
              == seg).astype(jnp.float32)
    s = jnp.dot(onehot, h, preferred_element_type=jnp.float32)
    cnt = jnp.sum(onehot, axis=1, keepdims=True)
    pool_ref[...] = s / jnp.maximum(cnt, 1.0)


def _post_call(a0, a1, h0, h1, dinv, w0, w1, b, g, be, batch):
    return pl.pallas_call(
        _post_body,
        out_shape=jax.ShapeDtypeStruct((_G, _H), jnp.float32),
    )(a0, a1, h0, h1, dinv, w0, w1, b, g, be, batch)


def _head_body(hc_ref, hs_ref, w1_ref, b1_ref, w2_ref, b2_ref, out_ref):
    x = jnp.concatenate([hc_ref[...], hs_ref[...]], axis=1)
    x = jnp.maximum(
        jnp.dot(x, w1_ref[...], preferred_element_type=jnp.float32)
        + b1_ref[...], 0.0)
    out_ref[...] = (jnp.dot(x, w2_ref[...], preferred_element_type=jnp.float32)
                    + b2_ref[...])


def _head_call(hc, hs, w1, b1, w2, b2):
    return pl.pallas_call(
        _head_body,
        out_shape=jax.ShapeDtypeStruct((_G, 2), jnp.float32),
    )(hc, hs, w1, b1, w2, b2)


def kernel(x_c, edge_index_c, batch_c, x_s, edge_index_s, batch_s,
           Wc1, bc1, gc1, betac1, Wc2, bc2, gc2, betac2,
           Ws1, bs1, gs1, betas1, Ws2, bs2, gs2, betas2,
           W_fc1, b_fc1, W_fc2, b_fc2):
    src_c, dst_c = edge_index_c[0].astype(jnp.int32), edge_index_c[1].astype(jnp.int32)
    src_s, dst_s = edge_index_s[0].astype(jnp.int32), edge_index_s[1].astype(jnp.int32)
    src_c2, dst_c2 = src_c.reshape(_E // _W, _W), dst_c.reshape(_E // _W, _W)
    src_s2, dst_s2 = src_s.reshape(_E // _W, _W), dst_s.reshape(_E // _W, _W)

    deg_parts = _deg_call(dst_c, dst_s)
    dinv2, xsc, xss = _prep_call(deg_parts, x_c, x_s)
    dinv_c = dinv2[0, :_N, None]
    dinv_s = dinv2[1, :_N, None]

    agg1_c, agg1_s = _conv1_call(src_c2, dst_c2, src_s2, dst_s2, xsc, xss)

    hc0, hc1 = _mid_call(agg1_c, xsc, dinv_c, Wc1, bc1.reshape(1, _H),
                         gc1.reshape(1, _H), betac1.reshape(1, _H))
    hs0, hs1 = _mid_call(agg1_s, xss, dinv_s, Ws1, bs1.reshape(1, _H),
                         gs1.reshape(1, _H), betas1.reshape(1, _H))

    ac0, ac1, as0, as1 = _conv2_call(src_c2, dst_c2, src_s2, dst_s2,
                                     hc0, hc1, hs0, hs1)

    pool_c = _post_call(ac0, ac1, hc0, hc1, dinv_c,
                        Wc2[:_D], Wc2[_D:], bc2.reshape(1, _H),
                        gc2.reshape(1, _H), betac2.reshape(1, _H),
                        batch_c.astype(jnp.int32).reshape(1, _N))
    pool_s = _post_call(as0, as1, hs0, hs1, dinv_s,
                        Ws2[:_D], Ws2[_D:], bs2.reshape(1, _H),
                        gs2.reshape(1, _H), betas2.reshape(1, _H),
                        batch_s.astype(jnp.int32).reshape(1, _N))

    return _head_call(pool_c, pool_s, W_fc1, b_fc1.reshape(1, _H),
                      W_fc2, b_fc2.reshape(1, 2))
